# baseline (speedup 1.0000x reference)
import functools

import numpy as np
import jax
import jax.numpy as jnp
from jax import lax
from jax.experimental import pallas as pl
from jax.experimental.pallas import tpu as pltpu
from jax.experimental.pallas import tpu_sc as plsc

_LENS = (2048, 512, 1024, 1536, 768, 1280, 256, 768)
_B = len(_LENS)
_OFFS = tuple(int(x) for x in np.concatenate([[0], np.cumsum(_LENS)]))
_TOTAL = _OFFS[-1]
_D = 256
_P = 4
_NW = 32
_CHUNK = tuple(l // _NW for l in _LENS)

_mesh = plsc.VectorSubcoreMesh(core_axis_name="c", subcore_axis_name="s")


@functools.partial(
    pl.kernel,
    out_type=(
        jax.ShapeDtypeStruct((_TOTAL + _B, _D), jnp.float32),
        jax.ShapeDtypeStruct((_TOTAL + _B, _P), jnp.float32),
    ),
    mesh=_mesh,
    scratch_types=[
        pltpu.VMEM((1, 16), jnp.float32),
        pltpu.SemaphoreType.DMA,
    ],
    compiler_params=pltpu.CompilerParams(use_tc_tiling_on_sc=False),
)
def _interleave_sc(q_hbm, pos_hbm, bg_hbm, outq_hbm, outp_hbm, zbuf, sem):
    wid = lax.axis_index("s") * 2 + lax.axis_index("c")

    copies = []
    for b in range(_B):
        c = _CHUNK[b]
        src = _OFFS[b] + wid * c
        copies.append(
            pltpu.async_copy(
                q_hbm.at[pl.ds(src, c)], outq_hbm.at[pl.ds(src + b, c)], sem
            )
        )

    for b in range(_B):

        @pl.when(wid == b)
        def _(b=b):
            pltpu.sync_copy(
                bg_hbm.at[pl.ds(b, 1)], outq_hbm.at[pl.ds(_OFFS[b + 1] + b, 1)]
            )

    for b in range(_B):

        @pl.when(wid == _B + b)
        def _(b=b):
            s, e = _OFFS[b], _OFFS[b + 1]
            pltpu.sync_copy(
                pos_hbm.at[pl.ds(s, e - s)], outp_hbm.at[pl.ds(s + b, e - s)]
            )

    for b in range(_B):

        @pl.when(wid == 2 * _B + b)
        def _(b=b):
            zbuf[0] = jnp.zeros((16,), jnp.float32)
            pltpu.sync_copy(
                zbuf.at[:, pl.ds(0, _P)], outp_hbm.at[pl.ds(_OFFS[b + 1] + b, 1)]
            )

    for cp in copies:
        cp.wait()


def kernel(queries, query_positions, query_batch_offsets, background_queries):
    bg = background_queries.reshape(_B, _D)
    outq, outp = _interleave_sc(queries, query_positions, bg)
    new_offsets = query_batch_offsets + jnp.arange(
        _B + 1, dtype=query_batch_offsets.dtype
    )
    return outq, outp, new_offsets


# speedup vs baseline: 5.1248x; 5.1248x over previous
import functools

import numpy as np
import jax
import jax.numpy as jnp
from jax import lax
from jax.experimental import pallas as pl
from jax.experimental.pallas import tpu as pltpu
from jax.experimental.pallas import tpu_sc as plsc

_LENS = (2048, 512, 1024, 1536, 768, 1280, 256, 768)
_B = len(_LENS)
_OFFS = tuple(int(x) for x in np.concatenate([[0], np.cumsum(_LENS)]))
_TOTAL = _OFFS[-1]
_D = 256
_P = 4
_NW = 32
_CHUNK = tuple(l // _NW for l in _LENS)

_mesh = plsc.VectorSubcoreMesh(core_axis_name="c", subcore_axis_name="s")


_CH_OFF = tuple(int(x) for x in np.concatenate([[0], np.cumsum(_CHUNK)]))


@functools.partial(
    pl.kernel,
    out_type=(
        jax.ShapeDtypeStruct((_TOTAL + _B, _D), jnp.float32),
        jax.ShapeDtypeStruct((_TOTAL + _B, _P), jnp.float32),
    ),
    mesh=_mesh,
    scratch_types=[
        pltpu.VMEM((_TOTAL // _NW, _D), jnp.float32),
        pltpu.VMEM((1, 16), jnp.float32),
        pltpu.SemaphoreType.DMA((_B,)),
        pltpu.SemaphoreType.DMA,
    ],
    compiler_params=pltpu.CompilerParams(use_tc_tiling_on_sc=False),
)
def _interleave_sc(q_hbm, pos_hbm, bg_hbm, outq_hbm, outp_hbm, qbuf, zbuf, lsem, ssem):
    wid = lax.axis_index("s") * 2 + lax.axis_index("c")

    loads = []
    for b in range(_B):
        c = _CHUNK[b]
        src = _OFFS[b] + wid * c
        loads.append(
            pltpu.async_copy(
                q_hbm.at[pl.ds(src, c)], qbuf.at[pl.ds(_CH_OFF[b], c)], lsem.at[b]
            )
        )
    copies = []
    for b in range(_B):
        c = _CHUNK[b]
        dst = _OFFS[b] + b + wid * c
        loads[b].wait()
        copies.append(
            pltpu.async_copy(
                qbuf.at[pl.ds(_CH_OFF[b], c)], outq_hbm.at[pl.ds(dst, c)], ssem
            )
        )

    for b in range(_B):

        @pl.when(wid == b)
        def _(b=b):
            pltpu.sync_copy(
                bg_hbm.at[pl.ds(b, 1)], outq_hbm.at[pl.ds(_OFFS[b + 1] + b, 1)]
            )

    for b in range(_B):

        @pl.when(wid == _B + b)
        def _(b=b):
            s, e = _OFFS[b], _OFFS[b + 1]
            pltpu.sync_copy(
                pos_hbm.at[pl.ds(s, e - s)], outp_hbm.at[pl.ds(s + b, e - s)]
            )

    for b in range(_B):

        @pl.when(wid == 2 * _B + b)
        def _(b=b):
            zbuf[0] = jnp.zeros((16,), jnp.float32)
            pltpu.sync_copy(
                zbuf.at[:, pl.ds(0, _P)], outp_hbm.at[pl.ds(_OFFS[b + 1] + b, 1)]
            )

    for cp in copies:
        cp.wait()


def kernel(queries, query_positions, query_batch_offsets, background_queries):
    bg = background_queries.reshape(_B, _D)
    outq, outp = _interleave_sc(queries, query_positions, bg)
    new_offsets = query_batch_offsets + jnp.arange(
        _B + 1, dtype=query_batch_offsets.dtype
    )
    return outq, outp, new_offsets


# speedup vs baseline: 7.1704x; 1.3992x over previous
import functools

import numpy as np
import jax
import jax.numpy as jnp
from jax import lax
from jax.experimental import pallas as pl
from jax.experimental.pallas import tpu as pltpu
from jax.experimental.pallas import tpu_sc as plsc

_LENS = (2048, 512, 1024, 1536, 768, 1280, 256, 768)
_B = len(_LENS)
_OFFS = tuple(int(x) for x in np.concatenate([[0], np.cumsum(_LENS)]))
_TOTAL = _OFFS[-1]
_D = 256
_P = 4
_NW = 32
_NV = _D // 16

_IS = tuple(_OFFS[b] + (8 if b else 0) for b in range(_B))
_NT = tuple((_OFFS[b + 1] - _IS[b]) // 8 for b in range(_B))
_MT = tuple(-(-n // _NW) for n in _NT)
_REG = tuple(8 * m + (8 if b else 0) for b, m in enumerate(_MT))
_RB = tuple(int(x) for x in np.concatenate([[0], np.cumsum(_REG)]))

_mesh = plsc.VectorSubcoreMesh(core_axis_name="c", subcore_axis_name="s")


@functools.partial(
    pl.kernel,
    out_type=jax.ShapeDtypeStruct((_TOTAL + _B, _D), jnp.float32),
    mesh=_mesh,
    scratch_types=[
        pltpu.VMEM((_RB[-1], _D), jnp.float32),
        pltpu.VMEM((16, _D), jnp.float32),
        pltpu.VMEM((8, _D), jnp.float32),
        pltpu.VMEM((_B, _D), jnp.float32),
        pltpu.SemaphoreType.DMA((_B,)),
        pltpu.SemaphoreType.DMA,
    ],
)
def _interleave_q_sc(q_hbm, bg_hbm, outq_hbm, buf, jbuf, obuf, bgbuf, lsem, ssem):
    wid = lax.axis_index("s") * 2 + lax.axis_index("c")

    starts = []
    loads = []
    for b in range(_B):
        m8 = 8 * _MT[b]
        t0 = jnp.minimum(wid * _MT[b], _NT[b] - _MT[b])
        a = _IS[b] + 8 * t0
        starts.append(a)
        src, n = (a, m8) if b == 0 else (a - 8, m8 + 8)
        loads.append(
            pltpu.async_copy(
                q_hbm.at[pl.ds(src, n)], buf.at[pl.ds(_RB[b], n)], lsem.at[b]
            )
        )

    for b in range(_B):

        @pl.when(wid == b)
        def _(b=b):
            if b >= 1:
                pltpu.sync_copy(
                    q_hbm.at[pl.ds(_OFFS[b + 1] - 8, 8)], jbuf.at[pl.ds(0, 8)]
                )
            if b < _B - 1:
                pltpu.sync_copy(
                    q_hbm.at[pl.ds(_OFFS[b + 1], 8)], jbuf.at[pl.ds(8, 8)]
                )
            pltpu.sync_copy(bg_hbm, bgbuf)
            for j in range(8):
                for k in range(_NV):
                    sl = pl.ds(16 * k, 16)
                    if j < b:
                        obuf[j, sl] = jbuf[8 - b + j, sl]
                    elif j == b:
                        obuf[j, sl] = bgbuf[b, sl]
                    else:
                        obuf[j, sl] = jbuf[8 + j - b - 1, sl]
            pltpu.sync_copy(obuf, outq_hbm.at[pl.ds(_OFFS[b + 1], 8)])

    stores = []
    for b in range(_B):
        m8 = 8 * _MT[b]
        loads[b].wait()
        if b > 0:
            pad = 8 - b

            def _shift(i, _, b=b, pad=pad):
                for k in range(_NV):
                    sl = pl.ds(16 * k, 16)
                    buf[_RB[b] + i, sl] = buf[_RB[b] + pad + i, sl]
                return _

            lax.fori_loop(0, m8, _shift, 0, unroll=4)
        stores.append(
            pltpu.async_copy(
                buf.at[pl.ds(_RB[b], m8)], outq_hbm.at[pl.ds(starts[b], m8)], ssem
            )
        )
    for cp in stores:
        cp.wait()


def _pos_tc_body(pos_ref, out_ref):
    zero = jnp.zeros((1, _P), jnp.float32)
    for b in range(_B):
        out_ref[pl.ds(_OFFS[b] + b, _LENS[b]), :] = pos_ref[
            pl.ds(_OFFS[b], _LENS[b]), :
        ]
        out_ref[pl.ds(_OFFS[b + 1] + b, 1), :] = zero


_pos_tc = pl.pallas_call(
    _pos_tc_body,
    out_shape=jax.ShapeDtypeStruct((_TOTAL + _B, _P), jnp.float32),
)


def kernel(queries, query_positions, query_batch_offsets, background_queries):
    bg = background_queries.reshape(_B, _D)
    outq = _interleave_q_sc(queries, bg)
    outp = _pos_tc(query_positions)
    new_offsets = query_batch_offsets + jnp.arange(
        _B + 1, dtype=query_batch_offsets.dtype
    )
    return outq, outp, new_offsets


# speedup vs baseline: 7.3277x; 1.0219x over previous
import functools

import numpy as np
import jax
import jax.numpy as jnp
from jax import lax
from jax.experimental import pallas as pl
from jax.experimental.pallas import tpu as pltpu
from jax.experimental.pallas import tpu_sc as plsc

_LENS = (2048, 512, 1024, 1536, 768, 1280, 256, 768)
_B = len(_LENS)
_OFFS = tuple(int(x) for x in np.concatenate([[0], np.cumsum(_LENS)]))
_TOTAL = _OFFS[-1]
_D = 256
_P = 4
_NW = 32
_NV = _D // 16

_IS = tuple(_OFFS[b] + (8 if b else 0) for b in range(_B))
_NT = tuple((_OFFS[b + 1] - _IS[b]) // 8 for b in range(_B))
_MT = tuple(-(-n // _NW) for n in _NT)
_REG = tuple(8 * m + (8 if b else 0) for b, m in enumerate(_MT))
_RB = tuple(int(x) for x in np.concatenate([[0], np.cumsum(_REG)]))

_mesh = plsc.VectorSubcoreMesh(core_axis_name="c", subcore_axis_name="s")


@functools.partial(
    pl.kernel,
    out_type=jax.ShapeDtypeStruct((_TOTAL + _B, _D), jnp.float32),
    mesh=_mesh,
    scratch_types=[
        pltpu.VMEM((_RB[-1], _D), jnp.float32),
        pltpu.VMEM((24, _D), jnp.float32),
        pltpu.VMEM((8, _D), jnp.float32),
        pltpu.VMEM((_B, _D), jnp.float32),
        pltpu.SemaphoreType.DMA((_B,)),
        pltpu.SemaphoreType.DMA,
    ],
)
def _interleave_q_sc(q_hbm, bg_hbm, outq_hbm, buf, jbuf, obuf, bgbuf, lsem, ssem):
    wid = lax.axis_index("s") * 2 + lax.axis_index("c")

    starts = []
    loads = []
    for b in range(_B):
        m8 = 8 * _MT[b]
        t0 = jnp.minimum(wid * _MT[b], _NT[b] - _MT[b])
        a = _IS[b] + 8 * t0
        starts.append(a)
        src, n = (a, m8) if b == 0 else (a - 8, m8 + 8)
        loads.append(
            pltpu.async_copy(
                q_hbm.at[pl.ds(src, n)], buf.at[pl.ds(_RB[b], n)], lsem.at[b]
            )
        )


    @pl.when(wid < _B)
    def _():
        e = jnp.int32(_OFFS[_B])
        for b in range(_B - 2, -1, -1):
            e = jnp.where(wid == b, _OFFS[b + 1], e)
        e = pl.multiple_of(e, 8)
        pltpu.sync_copy(q_hbm.at[pl.ds(e - 8, 8)], jbuf.at[pl.ds(0, 8)])
        pltpu.sync_copy(bg_hbm, bgbuf)

        @pl.when(wid < _B - 1)
        def _():
            pltpu.sync_copy(q_hbm.at[pl.ds(e, 8)], jbuf.at[pl.ds(16, 8)])

        for k in range(_NV):
            jbuf[8, pl.ds(16 * k, 16)] = bgbuf[wid, pl.ds(16 * k, 16)]

        def _asm(j, carry):
            s = 8 - wid + j
            row = jnp.where(s < 9, s, s + 7)
            for k in range(_NV):
                sl = pl.ds(16 * k, 16)
                obuf[j, sl] = jbuf[row, sl]
            return carry

        lax.fori_loop(0, 8, _asm, 0)
        pltpu.sync_copy(obuf, outq_hbm.at[pl.ds(e, 8)])

    stores = []
    for b in range(_B):
        m8 = 8 * _MT[b]
        loads[b].wait()
        if b > 0:
            pad = 8 - b

            def _shift(i, carry, b=b, pad=pad):
                for k in range(_NV):
                    sl = pl.ds(16 * k, 16)
                    buf[_RB[b] + i, sl] = buf[_RB[b] + pad + i, sl]
                return carry

            lax.fori_loop(0, m8, _shift, 0)
        stores.append(
            pltpu.async_copy(
                buf.at[pl.ds(_RB[b], m8)], outq_hbm.at[pl.ds(starts[b], m8)], ssem
            )
        )
    for cp in stores:
        cp.wait()


def _pos_tc_body(pos_ref, out_ref):
    zero = jnp.zeros((1, _P), jnp.float32)
    for b in range(_B):
        out_ref[pl.ds(_OFFS[b] + b, _LENS[b]), :] = pos_ref[
            pl.ds(_OFFS[b], _LENS[b]), :
        ]
        out_ref[pl.ds(_OFFS[b + 1] + b, 1), :] = zero


_pos_tc = pl.pallas_call(
    _pos_tc_body,
    out_shape=jax.ShapeDtypeStruct((_TOTAL + _B, _P), jnp.float32),
)


def kernel(queries, query_positions, query_batch_offsets, background_queries):
    bg = background_queries.reshape(_B, _D)
    outq = _interleave_q_sc(queries, bg)
    outp = _pos_tc(query_positions)
    new_offsets = query_batch_offsets + jnp.arange(
        _B + 1, dtype=query_batch_offsets.dtype
    )
    return outq, outp, new_offsets
